# Initial kernel scaffold; baseline (speedup 1.0000x reference)
import jax
import jax.numpy as jnp
from jax.experimental import pallas as pl


def kernel(input, emb_weight, or_weight, clause_idx):
    raise NotImplementedError("write your pallas kernel here")


# speedup vs baseline: 8.0496x; 8.0496x over previous
import functools

import jax
import jax.numpy as jnp
from jax import lax
from jax.experimental import pallas as pl
from jax.experimental.pallas import tpu as pltpu
from jax.experimental.pallas import tpu_sc as plsc

_NV = 10000
_NC = 42000
_K = 3
_B = 128
_NSUB = 16
_GROUPS = 165
_CPW = _GROUPS * 16
_NCPAD = _CPW * _NSUB
_NPAD = _NCPAD - _NC
_LPW = _CPW * _K
_THRESH = float(_NC - 1 + _NPAD)

def _sat_body(emb_hbm, idx_hbm, w_hbm, out_hbm,
              table_v, idx_v, w_v, part_v, part_sh, all_v, out_v):
    cid = lax.axis_index("c")
    sid = lax.axis_index("s")
    base = sid * _LPW
    pltpu.sync_copy(emb_hbm.at[0], table_v)
    pltpu.sync_copy(idx_hbm.at[pl.ds(base, _LPW)], idx_v)
    pltpu.sync_copy(w_hbm.at[pl.ds(base, _LPW)], w_v)

    lanes3 = lax.iota(jnp.int32, 16) * 3

    def body(i, acc):
        gbase = i * (16 * _K)
        pre = jnp.full((16,), float(_K - 1), jnp.float32)
        for j in range(_K):
            off = lanes3 + (gbase + j)
            lit = plsc.load_gather(idx_v, [off])
            ev = plsc.load_gather(table_v, [lit])
            wv = plsc.load_gather(w_v, [off])
            pre = pre + wv * jnp.sign(ev)
        return acc + jnp.sign(pre)

    acc = lax.fori_loop(0, _GROUPS, body, jnp.zeros((16,), jnp.float32))
    part_v[...] = acc
    pltpu.sync_copy(part_v, part_sh.at[sid])
    plsc.subcore_barrier()

    @pl.when(jnp.logical_and(cid == 0, sid == 0))
    def _finish():
        pltpu.sync_copy(part_sh, all_v)
        tot = all_v[0]
        for r in range(1, _NSUB):
            tot = tot + all_v[r]
        total = jnp.sum(tot)
        res = jnp.sign(total - _THRESH)
        resv = jnp.full((16,), res, jnp.float32)
        for k in range(_B // 16):
            out_v[pl.ds(k * 16, 16)] = resv
        pltpu.sync_copy(out_v, out_hbm)


@functools.lru_cache(maxsize=1)
def _build():
    mesh = plsc.VectorSubcoreMesh(
        core_axis_name="c", subcore_axis_name="s",
        num_cores=2, num_subcores=_NSUB,
    )
    return pl.kernel(
        _sat_body,
        out_type=jax.ShapeDtypeStruct((_B,), jnp.float32),
        mesh=mesh,
        compiler_params=pltpu.CompilerParams(needs_layout_passes=False),
        scratch_types=[
            pltpu.VMEM((_NV,), jnp.float32),
            pltpu.VMEM((_LPW,), jnp.int32),
            pltpu.VMEM((_LPW,), jnp.float32),
            pltpu.VMEM((_NSUB,), jnp.float32),
            pltpu.VMEM_SHARED((_NSUB, 16), jnp.float32),
            pltpu.VMEM((_NSUB, 16), jnp.float32),
            pltpu.VMEM((_B,), jnp.float32),
        ],
    )


def kernel(input, emb_weight, or_weight, clause_idx):
    del input
    pad = (_NCPAD - _NC) * _K
    idx_flat = jnp.pad(clause_idx.reshape(-1), (0, pad))
    w_flat = jnp.pad(or_weight.reshape(-1), (0, pad))
    return _build()(emb_weight, idx_flat, w_flat)
